# Initial kernel scaffold; baseline (speedup 1.0000x reference)
import jax
import jax.numpy as jnp
from jax.experimental import pallas as pl


def kernel(feat, W0, b0, W1, b1, a_att, r_att, edges_set, label_idx):
    raise NotImplementedError("write your pallas kernel here")


# speedup vs baseline: 2.4655x; 2.4655x over previous
import functools

import jax
import jax.numpy as jnp
from jax import lax
from jax.experimental import pallas as pl
from jax.experimental.pallas import tpu as pltpu
from jax.experimental.pallas import tpu_sc as plsc

N = 10000
E = 160000
S = 2
D = 128
NC = 2
NS = 16
L = 16
BATCH = 64
NB = 160
EPT = NB * BATCH
EPAD = NS * EPT
DUMMY = N
N_ACC = 10240
RPT = N_ACC // NS
PK = N_ACC // D
PKT = PK // NS
LBL = 1000
LBL_PAD = 1024
LPW = LBL_PAD // (NC * NS)

_mesh = lambda: plsc.VectorSubcoreMesh(core_axis_name="c", subcore_axis_name="s")


def _agg_body(table, srcs, dsts, zrows, apad,
              agg_out,
              srcv, dstv, rowb, hist, degv, attb, acc, hists_sh, gsem):
    c = lax.axis_index("c")
    s = lax.axis_index("s")

    pltpu.sync_copy(zrows, rowb)
    for j in range(RPT // BATCH):
        pltpu.sync_copy(rowb, acc.at[pl.ds(s * RPT + j * BATCH, BATCH)])

    z16 = jnp.zeros((L,), jnp.float32)

    def zero_hist(g, carry):
        hist[pl.ds(g * L, L)] = z16
        return carry

    lax.fori_loop(0, N_ACC // L, zero_hist, 0)

    pltpu.sync_copy(apad, attb)
    av = attb[...]
    av = av - jnp.full((L,), jnp.max(av, axis=0))
    ev = jnp.exp(av)
    tot = jnp.sum(ev, axis=0)
    attv = ev / jnp.full((L,), tot)
    lanes = lax.iota(jnp.int32, L)
    att_c = jnp.sum(jnp.where(lanes == c, attv, 0.0), axis=0)
    att_splat = jnp.full((L,), att_c)

    plsc.subcore_barrier()

    base = (c * NS + s) * EPT
    ones16 = jnp.full((L,), 1.0, jnp.float32)

    def step(b, carry):
        off = base + b * BATCH
        pltpu.sync_copy(srcs.at[pl.ds(off, BATCH)], srcv)
        pltpu.sync_copy(dsts.at[pl.ds(off, BATCH)], dstv)
        pltpu.async_copy(table.at[srcv], rowb, gsem).wait()
        pltpu.sync_copy(rowb, acc.at[dstv], add=True)
        for k in range(BATCH // L):
            idx = dstv[pl.ds(k * L, L)]
            plsc.addupdate_scatter(hist, [idx], ones16)
        return carry

    lax.fori_loop(0, NB, step, 0)

    pltpu.sync_copy(hist, hists_sh.at[s])
    plsc.subcore_barrier()
    for r in range(NS):
        pltpu.sync_copy(hists_sh.at[r].at[pl.ds(s * RPT, RPT)],
                        hist.at[pl.ds(r * RPT, RPT)])
    for g in range(RPT // L):
        tot16 = hist[pl.ds(g * L, L)]
        for r in range(1, NS):
            tot16 = tot16 + hist[pl.ds(r * RPT + g * L, L)]
        degv[pl.ds(g * L, L)] = tot16

    for j in range(RPT // BATCH):
        r0 = s * RPT + j * BATCH
        pltpu.sync_copy(acc.at[pl.ds(r0, BATCH)], rowb)

        def scale_row(r, carry):
            nloc = j * BATCH + r
            sub = degv[pl.ds(nloc & -16, L)]
            d = jnp.sum(jnp.where(lanes == (nloc & 15), sub, 0.0), axis=0)
            dsplat = jnp.full((L,), d)
            inv = jnp.where(dsplat > 0, att_splat / dsplat,
                            jnp.zeros((L,), jnp.float32))
            for q in range(D // L):
                rowb[r, pl.ds(q * L, L)] = rowb[r, pl.ds(q * L, L)] * inv
            return carry

        lax.fori_loop(0, BATCH, scale_row, 0)
        pltpu.sync_copy(rowb, agg_out.at[c].at[pl.ds(r0, BATCH)])


def _sc_aggregate(table, srcs, dsts, zrows, apad):
    return pl.kernel(
        _agg_body,
        out_type=jax.ShapeDtypeStruct((S, N_ACC, D), jnp.float32),
        mesh=_mesh(),
        compiler_params=pltpu.CompilerParams(needs_layout_passes=False),
        scratch_types=[
            pltpu.VMEM((BATCH,), jnp.int32),
            pltpu.VMEM((BATCH,), jnp.int32),
            pltpu.VMEM((BATCH, D), jnp.float32),
            pltpu.VMEM((N_ACC,), jnp.float32),
            pltpu.VMEM((RPT,), jnp.float32),
            pltpu.VMEM((L,), jnp.float32),
            pltpu.VMEM_SHARED((N_ACC, D), jnp.float32),
            pltpu.VMEM_SHARED((NS, N_ACC), jnp.float32),
            pltpu.SemaphoreType.DMA,
        ],
    )(table, srcs, dsts, zrows, apad)


def _gather_body(x, lab, out, idxv, rows, sem):
    c = lax.axis_index("c")
    s = lax.axis_index("s")
    w = s * NC + c
    pltpu.sync_copy(lab.at[pl.ds(w * LPW, LPW)], idxv)
    pltpu.async_copy(x.at[idxv], rows, sem).wait()
    pltpu.sync_copy(rows, out.at[pl.ds(w * LPW, LPW)])


def _sc_gather(x, lab):
    return pl.kernel(
        _gather_body,
        out_type=jax.ShapeDtypeStruct((LBL_PAD, D), jnp.float32),
        mesh=_mesh(),
        scratch_types=[
            pltpu.VMEM((LPW,), jnp.int32),
            pltpu.VMEM((LPW, D), jnp.float32),
            pltpu.SemaphoreType.DMA,
        ],
    )(x, lab)


def _mm_body(x_ref, w_ref, b_ref, o_ref):
    o_ref[...] = (jnp.dot(x_ref[...], w_ref[...],
                          preferred_element_type=jnp.float32) + b_ref[...])


def _tc_project(feat_pad, W0, b0):
    M, K = feat_pad.shape
    BM = 512
    return pl.pallas_call(
        _mm_body,
        grid=(pl.cdiv(M, BM),),
        in_specs=[
            pl.BlockSpec((BM, K), lambda i: (i, 0)),
            pl.BlockSpec((K, D), lambda i: (0, 0)),
            pl.BlockSpec((1, D), lambda i: (0, 0)),
        ],
        out_specs=pl.BlockSpec((BM, D), lambda i: (i, 0)),
        out_shape=jax.ShapeDtypeStruct((M, D), jnp.float32),
    )(feat_pad, W0, b0[None, :])


def _combine(a0_ref, a1_ref):
    x = a0_ref[...] + a1_ref[...]
    x = jnp.where(x >= 0, x, 0.2 * x)
    nrm = jnp.sqrt(jnp.sum(x * x, axis=1, keepdims=True))
    return x / jnp.maximum(nrm, 1e-12)


def _comb_mm_body(a0_ref, a1_ref, w_ref, b_ref, o_ref):
    x = _combine(a0_ref, a1_ref)
    o_ref[...] = (jnp.dot(x, w_ref[...],
                          preferred_element_type=jnp.float32) + b_ref[...])


def _comb_body(a0_ref, a1_ref, o_ref):
    o_ref[...] = _combine(a0_ref, a1_ref)


def _tc_combine(agg, W=None, b=None):
    BM = 512
    grid = (pl.cdiv(N_ACC, BM),)
    common_in = [
        pl.BlockSpec((BM, D), lambda i: (i, 0)),
        pl.BlockSpec((BM, D), lambda i: (i, 0)),
    ]
    if W is not None:
        return pl.pallas_call(
            _comb_mm_body,
            grid=grid,
            in_specs=common_in + [
                pl.BlockSpec((D, D), lambda i: (0, 0)),
                pl.BlockSpec((1, D), lambda i: (0, 0)),
            ],
            out_specs=pl.BlockSpec((BM, D), lambda i: (i, 0)),
            out_shape=jax.ShapeDtypeStruct((N_ACC, D), jnp.float32),
        )(agg[0], agg[1], W, b[None, :])
    return pl.pallas_call(
        _comb_body,
        grid=grid,
        in_specs=common_in,
        out_specs=pl.BlockSpec((BM, D), lambda i: (i, 0)),
        out_shape=jax.ShapeDtypeStruct((N_ACC, D), jnp.float32),
    )(agg[0], agg[1])


def kernel(feat, W0, b0, W1, b1, a_att, r_att, edges_set, label_idx):
    colA = edges_set[:, :, 0]
    colB = edges_set[:, :, 1]
    pad = jnp.full((S, EPAD - E), DUMMY, dtype=jnp.int32)
    colA = jnp.concatenate([colA, pad], axis=1).reshape(S * EPAD)
    colB = jnp.concatenate([colB, pad], axis=1).reshape(S * EPAD)

    feat_pad = jnp.zeros((N_ACC, feat.shape[1]), jnp.float32).at[:N].set(feat)
    zrows = jnp.zeros((BATCH, D), jnp.float32)
    a_pad = jnp.full((L,), -1e30, jnp.float32).at[:S].set(a_att)
    r_pad = jnp.full((L,), -1e30, jnp.float32).at[:S].set(r_att)

    lab = jnp.full((LBL_PAD,), 0, jnp.int32).at[:LBL].set(label_idx)

    support0 = _tc_project(feat_pad, W0, b0)
    agg1 = _sc_aggregate(support0, colA, colB, zrows, a_pad)
    support1 = _tc_combine(agg1, W1, b1)

    agg2 = _sc_aggregate(support1, colB, colA, zrows, r_pad)
    x2 = _tc_combine(agg2)

    out = _sc_gather(x2, lab)
    return out[:LBL]
